# baseline (speedup 1.0000x reference)
import functools

import jax
import jax.numpy as jnp
from jax import lax
from jax.experimental import pallas as pl
from jax.experimental.pallas import tpu as pltpu
from jax.experimental.pallas import tpu_sc as plsc

N_CODES = 8192
N_TOKENS = 8192
DIM = 256
TOK_BLK = 1024
HALF = N_TOKENS // 2


def _argmin_body(x_ref, e_ref, sx_ref, se_ref, codes_ref):
    m = lax.dot_general(
        x_ref[...], e_ref[...], (((1,), (1,)), ((), ())),
        preferred_element_type=jnp.float32)
    d = (sx_ref[...] - 2.0 * m) + se_ref[...]
    mn = jnp.min(d, axis=1, keepdims=True)
    io = lax.broadcasted_iota(jnp.int32, d.shape, 1)
    idx = jnp.min(jnp.where(d == mn, io, jnp.int32(2**30)), axis=1,
                  keepdims=True)
    codes_ref[...] = idx


def _argmin_call(x_bf, emb_bf, sx, se, interpret=False):
    n = x_bf.shape[0]
    return pl.pallas_call(
        _argmin_body,
        grid=(n // TOK_BLK,),
        in_specs=[
            pl.BlockSpec((TOK_BLK, DIM), lambda i: (i, 0)),
            pl.BlockSpec((N_CODES, DIM), lambda i: (0, 0)),
            pl.BlockSpec((TOK_BLK, 1), lambda i: (i, 0)),
            pl.BlockSpec((1, N_CODES), lambda i: (0, 0)),
        ],
        out_specs=pl.BlockSpec((TOK_BLK, 1), lambda i: (i, 0)),
        out_shape=jax.ShapeDtypeStruct((n, 1), jnp.int32),
        interpret=interpret,
    )(x_bf, emb_bf, sx, se)


_SC_WORKERS = 32
_B_PER_W = HALF // _SC_WORKERS


def _sc_gather_count(emb, idx):
    mesh = plsc.VectorSubcoreMesh(core_axis_name="c", subcore_axis_name="s")

    @functools.partial(
        pl.kernel,
        out_type=[
            jax.ShapeDtypeStruct((HALF, DIM), jnp.float32),
            jax.ShapeDtypeStruct((2, N_CODES), jnp.float32),
        ],
        mesh=mesh,
        scratch_types=[
            pltpu.VMEM((_B_PER_W,), jnp.int32),
            pltpu.VMEM((_B_PER_W, DIM), jnp.float32),
            pltpu.VMEM((_B_PER_W,), jnp.float32),
            pltpu.VMEM((1024,), jnp.float32),
            pltpu.VMEM_SHARED((N_CODES,), jnp.float32),
            pltpu.SemaphoreType.DMA,
        ],
    )
    def k(emb_hbm, idx_hbm, quant_hbm, pcnt_hbm,
          idx_v, rows_v, ones_v, zero_v, cnt_sh, sem):
        nc = 2
        cid = lax.axis_index("c")
        sid = lax.axis_index("s")
        wid = sid * nc + cid
        base = wid * _B_PER_W
        pltpu.sync_copy(idx_hbm.at[pl.ds(base, _B_PER_W)], idx_v)
        pltpu.async_copy(emb_hbm.at[idx_v], rows_v, sem).wait()
        pltpu.sync_copy(rows_v, quant_hbm.at[pl.ds(base, _B_PER_W)])

        def fill16(ref, n, value):
            def body(i, carry):
                ref[pl.ds(i * 16, 16)] = jnp.full((16,), value, jnp.float32)
                return carry
            lax.fori_loop(0, n // 16, body, 0)

        fill16(ones_v, _B_PER_W, 1.0)

        @pl.when(sid == 0)
        def _():
            fill16(zero_v, 1024, 0.0)
            for t in range(N_CODES // 1024):
                pltpu.sync_copy(zero_v, cnt_sh.at[pl.ds(t * 1024, 1024)])

        plsc.subcore_barrier()
        pltpu.sync_copy(ones_v, cnt_sh.at[idx_v], add=True)
        plsc.subcore_barrier()

        @pl.when(sid == 0)
        def _():
            pltpu.sync_copy(cnt_sh, pcnt_hbm.at[cid])

    return k(emb, idx)


def _final_body(x_ref, q1_ref, q2_ref, pc1_ref, pc2_ref,
                quant_ref, diff_ref, ppl_ref):
    x1 = x_ref[pl.ds(0, HALF), :]
    x2 = x_ref[pl.ds(HALF, HALF), :]
    q1 = q1_ref[...]
    q2 = q2_ref[...]
    d1 = q1 - x1
    d2 = q2 - x2
    quant_ref[pl.ds(0, HALF), :] = x1 + d1
    quant_ref[pl.ds(HALF, HALF), :] = x2 + d2
    diff_ref[...] = ((jnp.sum(d1 * d1, keepdims=True)
                      + jnp.sum(d2 * d2, keepdims=True))
                     / jnp.float32(N_TOKENS * DIM))
    counts = (jnp.sum(pc1_ref[...], axis=0, keepdims=True)
              + jnp.sum(pc2_ref[...], axis=0, keepdims=True))
    avg = counts / jnp.sum(counts)
    ppl_ref[...] = -jnp.sum(avg * jnp.log(avg + 1e-5), keepdims=True)


def _final_call(x, q1, q2, pc1, pc2, interpret=False):
    return pl.pallas_call(
        _final_body,
        out_shape=[
            jax.ShapeDtypeStruct((N_TOKENS, DIM), jnp.float32),
            jax.ShapeDtypeStruct((1, 1), jnp.float32),
            jax.ShapeDtypeStruct((1, 1), jnp.float32),
        ],
        interpret=interpret,
    )(x, q1, q2, pc1, pc2)


def kernel(x, embedding):
    sx = jnp.sum(x * x, axis=1).reshape(N_TOKENS, 1)
    se = jnp.sum(embedding * embedding, axis=1).reshape(1, N_CODES)
    xb = x.astype(jnp.bfloat16)
    eb = embedding.astype(jnp.bfloat16)
    c1 = _argmin_call(xb[:HALF], eb, sx[:HALF], se)
    c2 = _argmin_call(xb[HALF:], eb, sx[HALF:], se)
    q1, p1 = _sc_gather_count(embedding, c1.reshape(HALF))
    q2, p2 = _sc_gather_count(embedding, c2.reshape(HALF))
    quant, diff, ppl = _final_call(x, q1, q2, p1, p2)
    codes = jnp.concatenate([c1, c2], axis=0)
    return quant, diff.reshape(()), codes, ppl.reshape(())


# speedup vs baseline: 1.0537x; 1.0537x over previous
import functools

import jax
import jax.numpy as jnp
from jax import lax
from jax.experimental import pallas as pl
from jax.experimental.pallas import tpu as pltpu
from jax.experimental.pallas import tpu_sc as plsc

N_CODES = 8192
N_TOKENS = 8192
DIM = 256
TOK_BLK = 1024
N_BLOCKS = N_TOKENS // TOK_BLK


def _argmin_body(x_ref, e_ref, sx_ref, se_ref, codes_ref):
    m = lax.dot_general(
        x_ref[...], e_ref[...], (((1,), (1,)), ((), ())),
        preferred_element_type=jnp.float32)
    d = (sx_ref[...] - 2.0 * m) + se_ref[...]
    mn = jnp.min(d, axis=1, keepdims=True)
    io = lax.broadcasted_iota(jnp.int32, d.shape, 1)
    idx = jnp.min(jnp.where(d == mn, io, jnp.int32(2**30)), axis=1,
                  keepdims=True)
    codes_ref[...] = idx


def _argmin_call(x_bf, emb_bf, sx, se, interpret=False):
    return pl.pallas_call(
        _argmin_body,
        grid=(N_BLOCKS,),
        in_specs=[
            pl.BlockSpec((TOK_BLK, DIM), lambda i: (i, 0)),
            pl.BlockSpec((N_CODES, DIM), lambda i: (0, 0)),
            pl.BlockSpec((TOK_BLK, 1), lambda i: (i, 0)),
            pl.BlockSpec((1, N_CODES), lambda i: (0, 0)),
        ],
        out_specs=pl.BlockSpec((TOK_BLK, 1), lambda i: (i, 0)),
        out_shape=jax.ShapeDtypeStruct((N_TOKENS, 1), jnp.int32),
        interpret=interpret,
    )(x_bf, emb_bf, sx, se)


_SC_WORKERS = 32
_B_PER_W = N_TOKENS // _SC_WORKERS


_IDX_CHUNK = 128
_N_CHUNKS = _B_PER_W // _IDX_CHUNK


def _sc_gather_count(emb, idx):
    mesh = plsc.VectorSubcoreMesh(core_axis_name="c", subcore_axis_name="s")

    @functools.partial(
        pl.kernel,
        out_type=[
            jax.ShapeDtypeStruct((N_TOKENS, DIM), jnp.float32),
            jax.ShapeDtypeStruct((2, N_CODES), jnp.float32),
        ],
        mesh=mesh,
        scratch_types=[
            pltpu.VMEM((_N_CHUNKS, _IDX_CHUNK), jnp.int32),
            pltpu.VMEM((_B_PER_W, DIM), jnp.float32),
            pltpu.VMEM((_IDX_CHUNK,), jnp.float32),
            pltpu.VMEM((1024,), jnp.float32),
            pltpu.VMEM_SHARED((N_CODES,), jnp.float32),
            pltpu.SemaphoreType.DMA,
        ],
    )
    def k(emb_hbm, idx_hbm, quant_hbm, pcnt_hbm,
          idx_v, rows_v, ones_v, zero_v, cnt_sh, sem):
        nc = 2
        cid = lax.axis_index("c")
        sid = lax.axis_index("s")
        wid = sid * nc + cid
        base = wid * _B_PER_W
        for j in range(_N_CHUNKS):
            pltpu.sync_copy(idx_hbm.at[pl.ds(base + j * _IDX_CHUNK,
                                             _IDX_CHUNK)], idx_v.at[j])
        cps = [
            pltpu.async_copy(emb_hbm.at[idx_v.at[j]],
                             rows_v.at[pl.ds(j * _IDX_CHUNK, _IDX_CHUNK)],
                             sem)
            for j in range(_N_CHUNKS)
        ]
        for c in cps:
            c.wait()
        pltpu.sync_copy(rows_v, quant_hbm.at[pl.ds(base, _B_PER_W)])

        def fill16(ref, n, value):
            def body(i, carry):
                ref[pl.ds(i * 16, 16)] = jnp.full((16,), value, jnp.float32)
                return carry
            lax.fori_loop(0, n // 16, body, 0)

        fill16(ones_v, _IDX_CHUNK, 1.0)

        @pl.when(sid == 0)
        def _():
            fill16(zero_v, 1024, 0.0)
            for t in range(N_CODES // 1024):
                pltpu.sync_copy(zero_v, cnt_sh.at[pl.ds(t * 1024, 1024)])

        plsc.subcore_barrier()
        for j in range(_N_CHUNKS):
            pltpu.sync_copy(ones_v, cnt_sh.at[idx_v.at[j]], add=True)
        plsc.subcore_barrier()

        @pl.when(sid == 0)
        def _():
            pltpu.sync_copy(cnt_sh, pcnt_hbm.at[cid])

    return k(emb, idx)


def _final_body(x_ref, q_ref, pc_ref, quant_ref, diff_ref, ppl_ref):
    x = x_ref[...]
    q = q_ref[...]
    dlt = q - x
    quant_ref[...] = x + dlt
    diff_ref[...] = (jnp.sum(dlt * dlt, keepdims=True)
                     / jnp.float32(N_TOKENS * DIM))
    counts = jnp.sum(pc_ref[...], axis=0, keepdims=True)
    avg = counts / jnp.sum(counts)
    ppl_ref[...] = -jnp.sum(avg * jnp.log(avg + 1e-5), keepdims=True)


def _final_call(x, q, pcnt, interpret=False):
    return pl.pallas_call(
        _final_body,
        out_shape=[
            jax.ShapeDtypeStruct((N_TOKENS, DIM), jnp.float32),
            jax.ShapeDtypeStruct((1, 1), jnp.float32),
            jax.ShapeDtypeStruct((1, 1), jnp.float32),
        ],
        interpret=interpret,
    )(x, q, pcnt)


def kernel(x, embedding):
    sx = jnp.sum(x * x, axis=1).reshape(N_TOKENS, 1)
    se = jnp.sum(embedding * embedding, axis=1).reshape(1, N_CODES)
    codes = _argmin_call(x.astype(jnp.bfloat16),
                         embedding.astype(jnp.bfloat16),
                         sx, se)
    q, pcnt = _sc_gather_count(embedding, codes.reshape(N_TOKENS))
    quant, diff, ppl = _final_call(x, q, pcnt)
    return quant, diff.reshape(()), codes, ppl.reshape(())


# speedup vs baseline: 1.0550x; 1.0012x over previous
import functools

import jax
import jax.numpy as jnp
from jax import lax
from jax.experimental import pallas as pl
from jax.experimental.pallas import tpu as pltpu
from jax.experimental.pallas import tpu_sc as plsc

N_CODES = 8192
N_TOKENS = 8192
DIM = 256
TOK_BLK = 1024
N_BLOCKS = N_TOKENS // TOK_BLK


def _argmin_body(x_ref, e_ref, sx_ref, se_ref, codes_ref):
    m = lax.dot_general(
        x_ref[...], e_ref[...], (((1,), (1,)), ((), ())),
        preferred_element_type=jnp.float32)
    sx = sx_ref[...]
    se = se_ref[...]
    mn = jnp.min((sx - 2.0 * m) + se, axis=1, keepdims=True)
    io = lax.broadcasted_iota(jnp.int32, m.shape, 1)
    idx = jnp.min(jnp.where(((sx - 2.0 * m) + se) == mn, io,
                            jnp.int32(2**30)), axis=1, keepdims=True)
    codes_ref[...] = idx


def _argmin_call(x_bf, emb_bf, sx, se, interpret=False):
    return pl.pallas_call(
        _argmin_body,
        grid=(N_BLOCKS,),
        in_specs=[
            pl.BlockSpec((TOK_BLK, DIM), lambda i: (i, 0)),
            pl.BlockSpec((N_CODES, DIM), lambda i: (0, 0)),
            pl.BlockSpec((TOK_BLK, 1), lambda i: (i, 0)),
            pl.BlockSpec((1, N_CODES), lambda i: (0, 0)),
        ],
        out_specs=pl.BlockSpec((TOK_BLK, 1), lambda i: (i, 0)),
        out_shape=jax.ShapeDtypeStruct((N_TOKENS, 1), jnp.int32),
        interpret=interpret,
    )(x_bf, emb_bf, sx, se)


_SC_WORKERS = 32
_B_PER_W = N_TOKENS // _SC_WORKERS


_IDX_CHUNK = 128
_N_CHUNKS = _B_PER_W // _IDX_CHUNK


def _sc_gather_count(emb, idx):
    mesh = plsc.VectorSubcoreMesh(core_axis_name="c", subcore_axis_name="s")

    @functools.partial(
        pl.kernel,
        out_type=[
            jax.ShapeDtypeStruct((N_TOKENS, DIM), jnp.float32),
            jax.ShapeDtypeStruct((2, N_CODES), jnp.float32),
        ],
        mesh=mesh,
        scratch_types=[
            pltpu.VMEM((_N_CHUNKS, _IDX_CHUNK), jnp.int32),
            pltpu.VMEM((_B_PER_W, DIM), jnp.float32),
            pltpu.VMEM((_IDX_CHUNK,), jnp.float32),
            pltpu.VMEM((1024,), jnp.float32),
            pltpu.VMEM_SHARED((N_CODES,), jnp.float32),
            pltpu.SemaphoreType.DMA,
        ],
    )
    def k(emb_hbm, idx_hbm, quant_hbm, pcnt_hbm,
          idx_v, rows_v, ones_v, zero_v, cnt_sh, sem):
        nc = 2
        cid = lax.axis_index("c")
        sid = lax.axis_index("s")
        wid = sid * nc + cid
        base = wid * _B_PER_W
        for j in range(_N_CHUNKS):
            pltpu.sync_copy(idx_hbm.at[pl.ds(base + j * _IDX_CHUNK,
                                             _IDX_CHUNK)], idx_v.at[j])
        cps = [
            pltpu.async_copy(emb_hbm.at[idx_v.at[j]],
                             rows_v.at[pl.ds(j * _IDX_CHUNK, _IDX_CHUNK)],
                             sem)
            for j in range(_N_CHUNKS)
        ]
        for c in cps:
            c.wait()
        pltpu.sync_copy(rows_v, quant_hbm.at[pl.ds(base, _B_PER_W)])

        def fill16(ref, n, value):
            def body(i, carry):
                ref[pl.ds(i * 16, 16)] = jnp.full((16,), value, jnp.float32)
                return carry
            lax.fori_loop(0, n // 16, body, 0)

        fill16(ones_v, _IDX_CHUNK, 1.0)

        @pl.when(sid == 0)
        def _():
            fill16(zero_v, 1024, 0.0)
            for t in range(N_CODES // 1024):
                pltpu.sync_copy(zero_v, cnt_sh.at[pl.ds(t * 1024, 1024)])

        plsc.subcore_barrier()
        for j in range(_N_CHUNKS):
            pltpu.sync_copy(ones_v, cnt_sh.at[idx_v.at[j]], add=True)
        plsc.subcore_barrier()

        @pl.when(sid == 0)
        def _():
            pltpu.sync_copy(cnt_sh, pcnt_hbm.at[cid])

    return k(emb, idx)


def _final_body(x_ref, q_ref, pc_ref, quant_ref, diff_ref, ppl_ref):
    x = x_ref[...]
    q = q_ref[...]
    dlt = q - x
    quant_ref[...] = x + dlt
    diff_ref[...] = (jnp.sum(dlt * dlt, keepdims=True)
                     / jnp.float32(N_TOKENS * DIM))
    counts = jnp.sum(pc_ref[...], axis=0, keepdims=True)
    avg = counts / jnp.sum(counts)
    ppl_ref[...] = -jnp.sum(avg * jnp.log(avg + 1e-5), keepdims=True)


def _final_call(x, q, pcnt, interpret=False):
    return pl.pallas_call(
        _final_body,
        out_shape=[
            jax.ShapeDtypeStruct((N_TOKENS, DIM), jnp.float32),
            jax.ShapeDtypeStruct((1, 1), jnp.float32),
            jax.ShapeDtypeStruct((1, 1), jnp.float32),
        ],
        interpret=interpret,
    )(x, q, pcnt)


def kernel(x, embedding):
    sx = jnp.sum(x * x, axis=1).reshape(N_TOKENS, 1)
    se = jnp.sum(embedding * embedding, axis=1).reshape(1, N_CODES)
    codes = _argmin_call(x.astype(jnp.bfloat16),
                         embedding.astype(jnp.bfloat16),
                         sx, se)
    q, pcnt = _sc_gather_count(embedding, codes.reshape(N_TOKENS))
    quant, diff, ppl = _final_call(x, q, pcnt)
    return quant, diff.reshape(()), codes, ppl.reshape(())


# speedup vs baseline: 1.0738x; 1.0178x over previous
import functools

import jax
import jax.numpy as jnp
from jax import lax
from jax.experimental import pallas as pl
from jax.experimental.pallas import tpu as pltpu
from jax.experimental.pallas import tpu_sc as plsc

N_CODES = 8192
N_TOKENS = 8192
DIM = 256
TOK_BLK = 1024
N_BLOCKS = N_TOKENS // TOK_BLK


def _argmin_body(x_ref, e_ref, sx_ref, se_ref, codes_ref):
    m = lax.dot_general(
        x_ref[...], e_ref[...], (((1,), (1,)), ((), ())),
        preferred_element_type=jnp.float32)
    d = (sx_ref[...] - 2.0 * m) + se_ref[...]
    mn = jnp.min(d, axis=1, keepdims=True)
    io = lax.broadcasted_iota(jnp.int32, d.shape, 1)
    idx = jnp.min(jnp.where(d == mn, io, jnp.int32(2**30)), axis=1,
                  keepdims=True)
    codes_ref[...] = idx


def _argmin_call(x_bf, emb_bf, sx, se, interpret=False):
    return pl.pallas_call(
        _argmin_body,
        grid=(N_BLOCKS,),
        in_specs=[
            pl.BlockSpec((TOK_BLK, DIM), lambda i: (i, 0)),
            pl.BlockSpec((N_CODES, DIM), lambda i: (0, 0)),
            pl.BlockSpec((TOK_BLK, 1), lambda i: (i, 0)),
            pl.BlockSpec((1, N_CODES), lambda i: (0, 0)),
        ],
        out_specs=pl.BlockSpec((TOK_BLK, 1), lambda i: (i, 0)),
        out_shape=jax.ShapeDtypeStruct((N_TOKENS, 1), jnp.int32),
        interpret=interpret,
    )(x_bf, emb_bf, sx, se)


_SC_WORKERS = 32
_B_PER_W = N_TOKENS // _SC_WORKERS


_IDX_CHUNK = 128
_N_CHUNKS = _B_PER_W // _IDX_CHUNK


def _sc_gather_count(emb, idx):
    mesh = plsc.VectorSubcoreMesh(core_axis_name="c", subcore_axis_name="s")

    @functools.partial(
        pl.kernel,
        out_type=[
            jax.ShapeDtypeStruct((N_TOKENS, DIM), jnp.float32),
            jax.ShapeDtypeStruct((2, N_CODES), jnp.float32),
        ],
        mesh=mesh,
        scratch_types=[
            pltpu.VMEM((_N_CHUNKS, _IDX_CHUNK), jnp.int32),
            pltpu.VMEM((_B_PER_W, DIM), jnp.float32),
            pltpu.VMEM((_IDX_CHUNK,), jnp.float32),
            pltpu.VMEM((1024,), jnp.float32),
            pltpu.VMEM_SHARED((N_CODES,), jnp.float32),
            pltpu.SemaphoreType.DMA,
        ],
    )
    def k(emb_hbm, idx_hbm, quant_hbm, pcnt_hbm,
          idx_v, rows_v, ones_v, zero_v, cnt_sh, sem):
        nc = 2
        cid = lax.axis_index("c")
        sid = lax.axis_index("s")
        wid = sid * nc + cid
        base = wid * _B_PER_W
        for j in range(_N_CHUNKS):
            pltpu.sync_copy(idx_hbm.at[pl.ds(base + j * _IDX_CHUNK,
                                             _IDX_CHUNK)], idx_v.at[j])
        cps = [
            pltpu.async_copy(emb_hbm.at[idx_v.at[j]],
                             rows_v.at[pl.ds(j * _IDX_CHUNK, _IDX_CHUNK)],
                             sem)
            for j in range(_N_CHUNKS)
        ]
        for c in cps:
            c.wait()
        pltpu.sync_copy(rows_v, quant_hbm.at[pl.ds(base, _B_PER_W)])

        def fill16(ref, n, value):
            def body(i, carry):
                ref[pl.ds(i * 16, 16)] = jnp.full((16,), value, jnp.float32)
                return carry
            lax.fori_loop(0, n // 16, body, 0)

        fill16(ones_v, _IDX_CHUNK, 1.0)

        @pl.when(sid == 0)
        def _():
            fill16(zero_v, 1024, 0.0)
            for t in range(N_CODES // 1024):
                pltpu.sync_copy(zero_v, cnt_sh.at[pl.ds(t * 1024, 1024)])

        plsc.subcore_barrier()
        for j in range(_N_CHUNKS):
            pltpu.sync_copy(ones_v, cnt_sh.at[idx_v.at[j]], add=True)
        plsc.subcore_barrier()

        @pl.when(sid == 0)
        def _():
            pltpu.sync_copy(cnt_sh, pcnt_hbm.at[cid])

    return k(emb, idx)


def _final_body(x_ref, q_ref, pc_ref, diff_ref, ppl_ref):
    dlt = q_ref[...] - x_ref[...]
    diff_ref[...] = (jnp.sum(dlt * dlt, keepdims=True)
                     / jnp.float32(N_TOKENS * DIM))
    counts = jnp.sum(pc_ref[...], axis=0, keepdims=True)
    avg = counts / jnp.sum(counts)
    ppl_ref[...] = -jnp.sum(avg * jnp.log(avg + 1e-5), keepdims=True)


def _final_call(x, q, pcnt, interpret=False):
    return pl.pallas_call(
        _final_body,
        out_shape=[
            jax.ShapeDtypeStruct((1, 1), jnp.float32),
            jax.ShapeDtypeStruct((1, 1), jnp.float32),
        ],
        interpret=interpret,
    )(x, q, pcnt)


def kernel(x, embedding):
    sx = jnp.sum(x * x, axis=1).reshape(N_TOKENS, 1)
    se = jnp.sum(embedding * embedding, axis=1).reshape(1, N_CODES)
    codes = _argmin_call(x.astype(jnp.bfloat16),
                         embedding.astype(jnp.bfloat16),
                         sx, se)
    q, pcnt = _sc_gather_count(embedding, codes.reshape(N_TOKENS))
    diff, ppl = _final_call(x, q, pcnt)
    return q, diff.reshape(()), codes, ppl.reshape(())
